# Initial kernel scaffold; baseline (speedup 1.0000x reference)
import jax
import jax.numpy as jnp
from jax.experimental import pallas as pl


def kernel(logic_hidden, prompt_hidden, codebook_emb, W_init, W_q, W_k, W_v, slot_queries, W_slot_q, W_op_pre, W_gate, b_gate, W_stop, b_stop, W_ih, W_hh, b_ih, b_hh):
    raise NotImplementedError("write your pallas kernel here")


# speedup vs baseline: 2.2056x; 2.2056x over previous
import functools
import math

import jax
import jax.numpy as jnp
from jax.experimental import pallas as pl
from jax.experimental.pallas import tpu as pltpu

B = 16
S = 2048
H = 768
K = 8192
NSLOT = 9
STEPS = 4

S_T = 128
N_ST = S // S_T
K_T = 1024
N_KT = K // K_T

_RSQRT_H = 1.0 / math.sqrt(float(H))


def _init_kernel(p_ref, l_ref, w_init_ref, state_ref, psum, lsum):
    i = pl.program_id(0)

    @pl.when(i == 0)
    def _():
        psum[:] = jnp.zeros_like(psum)
        lsum[:] = jnp.zeros_like(lsum)

    psum[:] += jnp.sum(p_ref[:], axis=1)
    lsum[:] += jnp.sum(l_ref[:], axis=1)

    @pl.when(i == N_ST - 1)
    def _():
        ps = psum[:] * (1.0 / S)
        ls = lsum[:] * (1.0 / S)
        cat = jnp.concatenate([ps, ls], axis=-1)
        state_ref[:] = jnp.tanh(
            jax.lax.dot_general(cat, w_init_ref[:], (((1,), (1,)), ((), ())),
                                preferred_element_type=jnp.float32))


def _state_attend_kernel(state_ref, wq_ref, wk_ref, wv_ref, wop_ref, p_ref,
                         ctx_ref, oppre_ref, q_s, m_s, l_s, acc_s):
    i = pl.program_id(0)

    @pl.when(i == 0)
    def _():
        sq = jax.lax.dot_general(state_ref[:], wq_ref[:], (((1,), (1,)), ((), ())),
                                 preferred_element_type=jnp.float32)
        q_s[:] = jax.lax.dot_general(sq, wk_ref[:], (((1,), (0,)), ((), ())),
                                     preferred_element_type=jnp.float32) * _RSQRT_H
        m_s[:] = jnp.full_like(m_s, -jnp.inf)
        l_s[:] = jnp.zeros_like(l_s)
        acc_s[:] = jnp.zeros_like(acc_s)

    tile = p_ref[:]
    s = jax.lax.dot_general(q_s[:], tile, (((1,), (2,)), ((0,), (0,))),
                            preferred_element_type=jnp.float32)
    m_old = m_s[:]
    m_new = jnp.maximum(m_old, jnp.max(s, axis=1, keepdims=True))
    alpha = jnp.exp(m_old - m_new)
    p = jnp.exp(s - m_new)
    m_s[:] = m_new
    l_s[:] = alpha * l_s[:] + jnp.sum(p, axis=1, keepdims=True)
    acc_s[:] = alpha * acc_s[:] + jax.lax.dot_general(
        p, tile, (((1,), (1,)), ((0,), (0,))), preferred_element_type=jnp.float32)

    @pl.when(i == N_ST - 1)
    def _():
        raw = acc_s[:] / l_s[:]
        ctx = jax.lax.dot_general(raw, wv_ref[:], (((1,), (1,)), ((), ())),
                                  preferred_element_type=jnp.float32)
        ctx_ref[:] = ctx
        oppre_ref[:] = jax.lax.dot_general(ctx, wop_ref[:], (((1,), (1,)), ((), ())),
                                           preferred_element_type=jnp.float32)


def _slot_vq_kernel(ctx_ref, oppre_ref, state_ref, slotq_ref, wsq_ref, wk_ref,
                    wv_ref, wg_ref, bg_ref, ws1_ref, ws2_ref, bs_ref,
                    wih_ref, whh_ref, bih_ref, bhh_ref,
                    p_ref, c_ref,
                    newstate_ref, summary_ref, stoplogit_ref, stopprob_ref,
                    q_s, m_s, l_s, acc_s, g_s, bestv_s, beste_s):
    i = pl.program_id(0)

    @pl.when(i == 0)
    def _():
        seed = ctx_ref[:][:, None, :] + slotq_ref[:][None, :, :]
        q2 = jax.lax.dot_general(seed, wsq_ref[:], (((2,), (1,)), ((), ())),
                                 preferred_element_type=jnp.float32)
        q_s[:] = jax.lax.dot_general(q2, wk_ref[:], (((2,), (0,)), ((), ())),
                                     preferred_element_type=jnp.float32) * _RSQRT_H
        g_s[:] = jax.lax.dot_general(seed, wg_ref[:], (((2,), (1,)), ((), ())),
                                     preferred_element_type=jnp.float32)[:, :, 0] + bs0(bg_ref)
        m_s[:] = jnp.full_like(m_s, -jnp.inf)
        l_s[:] = jnp.zeros_like(l_s)
        acc_s[:] = jnp.zeros_like(acc_s)
        bestv_s[:] = jnp.full_like(bestv_s, -jnp.inf)
        beste_s[:] = jnp.zeros_like(beste_s)

    tile = p_ref[:]
    s = jax.lax.dot_general(q_s[:], tile, (((2,), (2,)), ((0,), (0,))),
                            preferred_element_type=jnp.float32)
    m_old = m_s[:]
    m_new = jnp.maximum(m_old, jnp.max(s, axis=2))
    alpha = jnp.exp(m_old - m_new)
    p = jnp.exp(s - m_new[:, :, None])
    m_s[:] = m_new
    l_s[:] = alpha * l_s[:] + jnp.sum(p, axis=2)
    acc_s[:] = alpha[:, :, None] * acc_s[:] + jax.lax.dot_general(
        p, tile, (((2,), (1,)), ((0,), (0,))), preferred_element_type=jnp.float32)

    @pl.when(i < N_KT)
    def _():
        ct = c_ref[:]
        logits = jax.lax.dot_general(oppre_ref[:], ct, (((1,), (1,)), ((), ())),
                                     preferred_element_type=jnp.float32)
        cn = 0.5 * jnp.sum(ct * ct, axis=1)
        val = logits - cn[None, :]
        tmax = jnp.max(val, axis=1, keepdims=True)
        iota = jax.lax.broadcasted_iota(jnp.int32, (B, K_T), 1)
        idx = jnp.min(jnp.where(val >= tmax, iota, K_T), axis=1)
        oh = (iota == idx[:, None]).astype(jnp.float32)
        row = jax.lax.dot_general(oh, ct, (((1,), (0,)), ((), ())),
                                  preferred_element_type=jnp.float32)
        better = tmax > bestv_s[:]
        bestv_s[:] = jnp.where(better, tmax, bestv_s[:])
        beste_s[:] = jnp.where(better, row, beste_s[:])

    @pl.when(i == N_ST - 1)
    def _():
        raw = acc_s[:] / l_s[:][:, :, None]
        slot_t = jax.lax.dot_general(raw, wv_ref[:], (((2,), (1,)), ((), ())),
                                     preferred_element_type=jnp.float32)
        probs = jax.nn.sigmoid(g_s[:])
        mask = probs >= 0.5
        any_used = jnp.sum(mask.astype(jnp.int32)) > 0
        pmax = jnp.max(probs, axis=1, keepdims=True)
        piota = jax.lax.broadcasted_iota(jnp.int32, (B, NSLOT), 1)
        top = jnp.min(jnp.where(probs >= pmax, piota, NSLOT), axis=1)
        fb = piota == top[:, None]
        mask_f = jnp.where(any_used, mask.astype(jnp.float32),
                           fb.astype(jnp.float32))
        denom = jnp.clip(jnp.sum(mask_f, axis=1, keepdims=True), 1.0, None)
        ssum = jnp.sum(slot_t * mask_f[:, :, None], axis=1) / denom
        msum = jnp.tanh(beste_s[:] + ssum)
        ctx = ctx_ref[:]
        stop = (jax.lax.dot_general(ctx, ws1_ref[:], (((1,), (1,)), ((), ())),
                                    preferred_element_type=jnp.float32)
                + jax.lax.dot_general(msum, ws2_ref[:], (((1,), (1,)), ((), ())),
                                      preferred_element_type=jnp.float32)
                + bs0(bs_ref))
        summary_ref[:] = msum
        stoplogit_ref[:] = stop
        stopprob_ref[:] = jax.nn.sigmoid(stop)
        gi = jax.lax.dot_general(msum, wih_ref[:], (((1,), (1,)), ((), ())),
                                 preferred_element_type=jnp.float32) + bih_ref[:]
        gh = jax.lax.dot_general(state_ref[:], whh_ref[:], (((1,), (1,)), ((), ())),
                                 preferred_element_type=jnp.float32) + bhh_ref[:]
        i_r, i_z, i_n = gi[:, :H], gi[:, H:2 * H], gi[:, 2 * H:]
        h_r, h_z, h_n = gh[:, :H], gh[:, H:2 * H], gh[:, 2 * H:]
        r = jax.nn.sigmoid(i_r + h_r)
        z = jax.nn.sigmoid(i_z + h_z)
        n = jnp.tanh(i_n + r * h_n)
        newstate_ref[:] = (1.0 - z) * n + z * state_ref[:]


def bs0(ref):
    return ref[0, 0]


def _chain_kernel(probs_ref, out_ref):
    hits = (probs_ref[:] >= 0.5).astype(jnp.int32)
    iota = jax.lax.broadcasted_iota(jnp.int32, (B, STEPS), 1)
    hmax = jnp.max(hits, axis=1, keepdims=True)
    first = jnp.min(jnp.where(hits >= hmax, iota, STEPS), axis=1, keepdims=True)
    cl = first + 1
    out_ref[:] = jnp.where(jnp.sum(hits, axis=1, keepdims=True) == 0,
                           jnp.full_like(cl, STEPS), cl)


def _full(shape):
    return pl.BlockSpec(shape, lambda i: (0,) * len(shape))


@jax.jit
def kernel(logic_hidden, prompt_hidden, codebook_emb, W_init, W_q, W_k, W_v,
           slot_queries, W_slot_q, W_op_pre, W_gate, b_gate, W_stop, b_stop,
           W_ih, W_hh, b_ih, b_hh):
    f32 = jnp.float32
    bg2 = b_gate.reshape(1, 1).astype(f32)
    bs2 = b_stop.reshape(1, 1).astype(f32)
    ws1 = W_stop[:, :H]
    ws2 = W_stop[:, H:]
    bih2 = b_ih.reshape(1, 3 * H)
    bhh2 = b_hh.reshape(1, 3 * H)

    state = pl.pallas_call(
        _init_kernel,
        grid=(N_ST,),
        in_specs=[
            pl.BlockSpec((B, S_T, H), lambda i: (0, i, 0)),
            pl.BlockSpec((B, S_T, H), lambda i: (0, i, 0)),
            _full((H, 2 * H)),
        ],
        out_specs=_full((B, H)),
        out_shape=jax.ShapeDtypeStruct((B, H), f32),
        scratch_shapes=[pltpu.VMEM((B, H), f32), pltpu.VMEM((B, H), f32)],
    )(prompt_hidden, logic_hidden, W_init)

    attend_a = pl.pallas_call(
        _state_attend_kernel,
        grid=(N_ST,),
        in_specs=[
            _full((B, H)), _full((H, H)), _full((H, H)), _full((H, H)),
            _full((H, H)),
            pl.BlockSpec((B, S_T, H), lambda i: (0, i, 0)),
        ],
        out_specs=[_full((B, H)), _full((B, H))],
        out_shape=[jax.ShapeDtypeStruct((B, H), f32),
                   jax.ShapeDtypeStruct((B, H), f32)],
        scratch_shapes=[pltpu.VMEM((B, H), f32), pltpu.VMEM((B, 1), f32),
                        pltpu.VMEM((B, 1), f32), pltpu.VMEM((B, H), f32)],
    )

    slot_vq = pl.pallas_call(
        _slot_vq_kernel,
        grid=(N_ST,),
        in_specs=[
            _full((B, H)), _full((B, H)), _full((B, H)), _full((NSLOT, H)),
            _full((H, H)), _full((H, H)), _full((H, H)),
            _full((1, H)), _full((1, 1)), _full((1, H)), _full((1, H)),
            _full((1, 1)), _full((3 * H, H)), _full((3 * H, H)),
            _full((1, 3 * H)), _full((1, 3 * H)),
            pl.BlockSpec((B, S_T, H), lambda i: (0, i, 0)),
            pl.BlockSpec((K_T, H), lambda i: (jnp.minimum(i, N_KT - 1), 0)),
        ],
        out_specs=[_full((B, H)), _full((B, H)), _full((B, 1)), _full((B, 1))],
        out_shape=[jax.ShapeDtypeStruct((B, H), f32),
                   jax.ShapeDtypeStruct((B, H), f32),
                   jax.ShapeDtypeStruct((B, 1), f32),
                   jax.ShapeDtypeStruct((B, 1), f32)],
        scratch_shapes=[pltpu.VMEM((B, NSLOT, H), f32),
                        pltpu.VMEM((B, NSLOT), f32),
                        pltpu.VMEM((B, NSLOT), f32),
                        pltpu.VMEM((B, NSLOT, H), f32),
                        pltpu.VMEM((B, NSLOT), f32),
                        pltpu.VMEM((B, 1), f32),
                        pltpu.VMEM((B, H), f32)],
    )

    summaries = []
    stop_logits = []
    stop_probs = []
    for _ in range(STEPS):
        state_ctx, op_pre = attend_a(state, W_q, W_k, W_v, W_op_pre,
                                     prompt_hidden)
        state, summ, slog, sprob = slot_vq(
            state_ctx, op_pre, state, slot_queries, W_slot_q, W_k, W_v,
            W_gate, bg2, ws1, ws2, bs2, W_ih, W_hh, bih2, bhh2,
            prompt_hidden, codebook_emb)
        summaries.append(summ)
        stop_logits.append(slog[:, 0])
        stop_probs.append(sprob[:, 0])

    stop_logits_t = jnp.stack(stop_logits, axis=1)
    stop_probs_t = jnp.stack(stop_probs, axis=1)
    summary_stack = jnp.stack(summaries, axis=1)

    chain = pl.pallas_call(
        _chain_kernel,
        grid=(1,),
        in_specs=[_full((B, STEPS))],
        out_specs=_full((B, 1)),
        out_shape=jax.ShapeDtypeStruct((B, 1), jnp.int32),
    )(stop_probs_t)

    return stop_logits_t, stop_probs_t, summary_stack, chain[:, 0]
